# baseline (speedup 1.0000x reference)
import functools

import jax
import jax.numpy as jnp
import numpy as np
from jax import lax
from jax.experimental import pallas as pl

B, S, D, H = 4, 8192, 768, 96
KEEP = max(1, int(S * 0.7))
BS = 1024

f32 = np.float32
_T_COEF = [7.853861595399531e-5, -8.010193625184903e-4, 5.188327685732524e-3,
           -2.685381193529856e-2, 1.128358514861418e-1, -3.761262582423300e-1,
           1.128379165726710e+0]
_P_COEF = [2.326819970068386e-2, -1.387039388740657e-1, 3.687424674597105e-1,
           -5.824733027278666e-1, 6.210004621745983e-1, -4.944515323274145e-1,
           3.404879937665872e-1, -2.741127028184656e-1, 5.638259427386472e-1]
_R_COEF = [-1.047766399936249e+1, 1.297719955372516e+1, -7.495518717768503e+0,
           2.921019019210786e+0, -1.015265279202700e+0, 4.218463358204948e-1,
           -2.820767439740514e-1, 5.641895067754075e-1]
_LOG2E = f32(1.4426950408889634)
_MAXLOG = f32(88.72283905206835)
_SQRT_HALF = f32(0.7071067811865476)


def _horner(y, coefs):
    acc = f32(coefs[0]) * y
    for c in coefs[1:-1]:
        acc = (acc + f32(c)) * y
    return acc + f32(coefs[-1])


def _gelu_exact(hp):
    a = (-_SQRT_HALF) * hp
    w = a * a
    small = f32(1.0) - a * _horner(w, _T_COEF)
    y = pl.reciprocal(w, approx=True)
    p = jnp.where(jnp.abs(a) < f32(2.0), _horner(y, _P_COEF), _horner(y, _R_COEF))
    z = jnp.exp2(_LOG2E * (-w))
    q = pl.reciprocal(jnp.abs(a), approx=True)
    large = p * (q * z)
    large = jnp.where((-w) < (-_MAXLOG), f32(0.0), large)
    large = jnp.where(a < f32(0.0), f32(2.0) - large, large)
    erfc_res = jnp.where(jnp.abs(a) < f32(1.0), small, large)
    return erfc_res * (f32(0.5) * hp)


def _sigmoid(t):
    return pl.reciprocal(f32(1.0) + jnp.exp2((-_LOG2E) * t), approx=True)


def _score_body(x_ref, W1_ref, b1_ref, W2_ref, b2_ref, s_ref):
    hpT = lax.dot_general(W1_ref[...], x_ref[...], (((0,), (1,)), ((), ())),
                          preferred_element_type=jnp.float32)
    hpT = hpT + b1_ref[...][:, None]
    hT = _gelu_exact(hpT)
    spT = lax.dot_general(W2_ref[...], hT, (((0,), (0,)), ((), ())),
                          preferred_element_type=jnp.float32)
    s_ref[...] = _sigmoid(spT + b2_ref[...][0])


@jax.jit
def _scores(x2, W1, b1, W2, b2):
    n = x2.shape[0]
    return pl.pallas_call(
        _score_body,
        grid=(n // BS,),
        in_specs=[
            pl.BlockSpec((BS, D), lambda i: (i, 0)),
            pl.BlockSpec((D, H), lambda i: (0, 0)),
            pl.BlockSpec((H,), lambda i: (0,)),
            pl.BlockSpec((H, 1), lambda i: (0, 0)),
            pl.BlockSpec((1,), lambda i: (0,)),
        ],
        out_specs=pl.BlockSpec((1, BS), lambda i: (0, i)),
        out_shape=jax.ShapeDtypeStruct((1, n), jnp.float32),
    )(x2, W1, b1, W2, b2)


def kernel(x, W1, b1, W2, b2):
    x2 = x.reshape(B * S, D)
    scores = _scores(x2, W1, b1, W2, b2).reshape(B, S)
    _, keep_indices = jax.lax.top_k(scores, KEEP)
    KPAD = 5760
    idx_pad = jnp.pad(keep_indices, ((0, 0), (0, KPAD - KEEP)))
    flat_idx = (idx_pad + (jnp.arange(B, dtype=jnp.int32) * S)[:, None]).reshape(-1)
    kept_pad = jnp.take(x2, flat_idx, axis=0, mode="clip")
    kept_pad = jax.lax.optimization_barrier(kept_pad)
    kept = jnp.stack([kept_pad[b * KPAD:b * KPAD + KEEP] for b in range(B)])
    return (kept, keep_indices)


# speedup vs baseline: 2.2880x; 2.2880x over previous
import functools

import jax
import jax.numpy as jnp
import numpy as np
from jax import lax
from jax.experimental import pallas as pl

B, S, D, H = 4, 8192, 768, 96
KEEP = max(1, int(S * 0.7))
BS = 1024

f32 = np.float32
_T_COEF = [7.853861595399531e-5, -8.010193625184903e-4, 5.188327685732524e-3,
           -2.685381193529856e-2, 1.128358514861418e-1, -3.761262582423300e-1,
           1.128379165726710e+0]
_P_COEF = [2.326819970068386e-2, -1.387039388740657e-1, 3.687424674597105e-1,
           -5.824733027278666e-1, 6.210004621745983e-1, -4.944515323274145e-1,
           3.404879937665872e-1, -2.741127028184656e-1, 5.638259427386472e-1]
_R_COEF = [-1.047766399936249e+1, 1.297719955372516e+1, -7.495518717768503e+0,
           2.921019019210786e+0, -1.015265279202700e+0, 4.218463358204948e-1,
           -2.820767439740514e-1, 5.641895067754075e-1]
_LOG2E = f32(1.4426950408889634)
_MAXLOG = f32(88.72283905206835)
_SQRT_HALF = f32(0.7071067811865476)


def _horner(y, coefs):
    acc = f32(coefs[0]) * y
    for c in coefs[1:-1]:
        acc = (acc + f32(c)) * y
    return acc + f32(coefs[-1])


def _gelu_exact(hp):
    a = (-_SQRT_HALF) * hp
    w = a * a
    small = f32(1.0) - a * _horner(w, _T_COEF)
    y = pl.reciprocal(w, approx=True)
    p = jnp.where(jnp.abs(a) < f32(2.0), _horner(y, _P_COEF), _horner(y, _R_COEF))
    z = jnp.exp2(_LOG2E * (-w))
    q = pl.reciprocal(jnp.abs(a), approx=True)
    large = p * (q * z)
    large = jnp.where((-w) < (-_MAXLOG), f32(0.0), large)
    large = jnp.where(a < f32(0.0), f32(2.0) - large, large)
    erfc_res = jnp.where(jnp.abs(a) < f32(1.0), small, large)
    return erfc_res * (f32(0.5) * hp)


def _sigmoid(t):
    return pl.reciprocal(f32(1.0) + jnp.exp2((-_LOG2E) * t), approx=True)


def _score_body(x_ref, W1_ref, b1_ref, W2_ref, b2_ref, s_ref):
    hpT = lax.dot_general(W1_ref[...], x_ref[...], (((0,), (1,)), ((), ())),
                          preferred_element_type=jnp.float32)
    hpT = hpT + b1_ref[...][:, None]
    hT = _gelu_exact(hpT)
    spT = lax.dot_general(W2_ref[...], hT, (((0,), (0,)), ((), ())),
                          preferred_element_type=jnp.float32)
    s_ref[...] = _sigmoid(spT + b2_ref[...][0])


@jax.jit
def _scores(x2, W1, b1, W2, b2):
    n = x2.shape[0]
    return pl.pallas_call(
        _score_body,
        grid=(n // BS,),
        in_specs=[
            pl.BlockSpec((BS, D), lambda i: (i, 0)),
            pl.BlockSpec((D, H), lambda i: (0, 0)),
            pl.BlockSpec((H,), lambda i: (0,)),
            pl.BlockSpec((H, 1), lambda i: (0, 0)),
            pl.BlockSpec((1,), lambda i: (0,)),
        ],
        out_specs=pl.BlockSpec((1, BS), lambda i: (0, i)),
        out_shape=jax.ShapeDtypeStruct((1, n), jnp.float32),
    )(x2, W1, b1, W2, b2)


def kernel(x, W1, b1, W2, b2):
    x2 = x.reshape(B * S, D)
    scores = _scores(x2, W1, b1, W2, b2).reshape(B, S)
    _, keep_indices = jax.lax.top_k(scores, KEEP)
    KPAD = 5760
    idx_pad = jnp.pad(keep_indices, ((0, 0), (0, KPAD - KEEP)))
    kept_pad = jnp.take_along_axis(x, idx_pad[:, :, None], axis=1)
    kept_pad = jax.lax.optimization_barrier(kept_pad)
    kept = kept_pad[:, :KEEP, :]
    return (kept, keep_indices)
